# Initial kernel scaffold; baseline (speedup 1.0000x reference)
import jax
import jax.numpy as jnp
from jax.experimental import pallas as pl


def kernel(solute_x, solute_edge_index, solute_edge_attr, solute_batch, solvent_x, solvent_edge_index, solvent_edge_attr, solvent_batch, params):
    raise NotImplementedError("write your pallas kernel here")


# speedup vs baseline: 1.9547x; 1.9547x over previous
import functools

import jax
import jax.numpy as jnp
from jax import lax
from jax.experimental import pallas as pl
from jax.experimental.pallas import tpu as pltpu
from jax.experimental.pallas import tpu_sc as plsc

N, E, B, H, EI, STEPS, S2S_STEPS = 4096, 8192, 128, 52, 10, 3, 2
HP = 64
NT = EI + 1
MW = 80
EB = 1024
TILE = 256
DP = 128
NW = 32
EPW = E // NW
CHUNK = 128

_f32 = jnp.float32


def _pad2(a, shape):
    return jnp.zeros(shape, _f32).at[: a.shape[0], : a.shape[1]].set(a)


def _prep_gather(p):
    en2w = p['en2_w']
    astack = jnp.zeros((HP, NT * HP), _f32)
    for i in range(EI):
        astack = astack.at[:H, i * HP:i * HP + H].set(en2w[:, i].reshape(H, H))
    astack = astack.at[:H, EI * HP:EI * HP + H].set(p['en2_b'].reshape(H, H))
    msg_w = jnp.zeros((HP, 2 * HP), _f32)
    msg_w = msg_w.at[:H, :H].set(p['msg_w'][:, :H])
    msg_w = msg_w.at[:H, HP:HP + H].set(p['msg_w'][:, H:2 * H])
    return {
        'lin0_w': _pad2(p['lin0_w'], (HP, HP)),
        'lin0_b': jnp.zeros((1, HP), _f32).at[0, :H].set(p['lin0_b']),
        'en1_w': _pad2(p['en1_w'], (16, 16)),
        'en1_b': jnp.zeros((1, 16), _f32).at[0, :EI].set(p['en1_b']),
        'astack': astack,
        'root_w': _pad2(p['root_w'], (HP, HP)),
        'conv_b': jnp.zeros((1, HP), _f32).at[0, :H].set(p['conv_b']),
        'msg_w': msg_w,
        'msg_b': jnp.zeros((1, HP), _f32).at[0, :H].set(p['msg_b']),
    }


def _prep_s2s(p):
    d = 2 * H
    wih_p = jnp.zeros((4 * DP, 2 * DP), _f32)
    whh_p = jnp.zeros((4 * DP, DP), _f32)
    bih_p = jnp.zeros((1, 4 * DP), _f32)
    bhh_p = jnp.zeros((1, 4 * DP), _f32)
    for g in range(4):
        wih_p = wih_p.at[g * DP:g * DP + d, :d].set(p['wih'][g * d:(g + 1) * d, :d])
        wih_p = wih_p.at[g * DP:g * DP + d, DP:DP + d].set(p['wih'][g * d:(g + 1) * d, d:2 * d])
        whh_p = whh_p.at[g * DP:g * DP + d, :d].set(p['whh'][g * d:(g + 1) * d, :])
        bih_p = bih_p.at[0, g * DP:g * DP + d].set(p['bih'][g * d:(g + 1) * d])
        bhh_p = bhh_p.at[0, g * DP:g * DP + d].set(p['bhh'][g * d:(g + 1) * d])
    return wih_p, whh_p, bih_p, bhh_p


def _prep_pred(p):
    d = 2 * H
    w1 = jnp.zeros((256, 4 * DP), _f32)
    for g in range(4):
        w1 = w1.at[:, g * DP:g * DP + d].set(p['pred1_w'][:, g * d:(g + 1) * d])
    w3 = jnp.zeros((DP, DP), _f32).at[0, :].set(p['pred3_w'][0])
    return w1, w3


def _pre_body(xu, xv, eau, eav, l0wu, l0bu, l0wv, l0bv, e1wu, e1bu, e1wv, e1bv,
              outu, outv, heu, hev):
    outu[...] = jax.nn.relu(
        jnp.dot(xu[...], l0wu[...].T, preferred_element_type=_f32) + l0bu[...])
    outv[...] = jax.nn.relu(
        jnp.dot(xv[...], l0wv[...].T, preferred_element_type=_f32) + l0bv[...])
    col = lax.broadcasted_iota(jnp.int32, (E, 16), 1)
    hu = jax.nn.relu(
        jnp.dot(eau[...], e1wu[...].T, preferred_element_type=_f32) + e1bu[...])
    heu[...] = jnp.where(col == EI, 1.0, hu)
    hv = jax.nn.relu(
        jnp.dot(eav[...], e1wv[...].T, preferred_element_type=_f32) + e1bv[...])
    hev[...] = jnp.where(col == EI, 1.0, hv)


def _tc_pre(xu, xv, eau, eav, pu, pv):
    return pl.pallas_call(
        _pre_body,
        out_shape=[jax.ShapeDtypeStruct((N, HP), _f32),
                   jax.ShapeDtypeStruct((N, HP), _f32),
                   jax.ShapeDtypeStruct((E, 16), _f32),
                   jax.ShapeDtypeStruct((E, 16), _f32)],
    )(xu, xv, eau, eav, pu['lin0_w'], pu['lin0_b'], pv['lin0_w'], pv['lin0_b'],
      pu['en1_w'], pu['en1_b'], pv['en1_w'], pv['en1_b'])


_SC_MESH = plsc.VectorSubcoreMesh(core_axis_name="c", subcore_axis_name="s")


@functools.partial(
    pl.kernel, mesh=_SC_MESH,
    out_type=[jax.ShapeDtypeStruct((E, HP), _f32),
              jax.ShapeDtypeStruct((E, HP), _f32)],
    scratch_types=[pltpu.VMEM((CHUNK,), jnp.int32),
                   pltpu.VMEM((CHUNK, HP), _f32),
                   pltpu.SemaphoreType.DMA],
    compiler_params=pltpu.CompilerParams(use_tc_tiling_on_sc=False),
)
def _sc_gather(tab_u, tab_v, src_u, src_v, gu, gv, idx_v, rows_v, sem):
    cid = lax.axis_index("c")
    sid = lax.axis_index("s")
    wid = sid * 2 + cid
    for tab, src, out in ((tab_u, src_u, gu), (tab_v, src_v, gv)):
        for ch in range(EPW // CHUNK):
            base = wid * EPW + ch * CHUNK
            pltpu.sync_copy(src.at[pl.ds(base, CHUNK)], idx_v)
            pltpu.async_copy(tab.at[idx_v], rows_v, sem).wait()
            pltpu.sync_copy(rows_v, out.at[pl.ds(base, CHUNK)])


def _msg_body(gu, heu, asu, gv, hev, asv, mu, mv):
    for g, he, astk, out in ((gu, heu, asu, mu), (gv, hev, asv, mv)):
        p = jnp.dot(g[...], astk[...], preferred_element_type=_f32)
        acc = jnp.zeros((EB, HP), _f32)
        hev_ = he[...]
        for i in range(NT):
            acc = acc + hev_[:, i:i + 1] * p[:, i * HP:(i + 1) * HP]
        col = lax.broadcasted_iota(jnp.int32, (EB, 16), 1)
        tail = jnp.where(col == 0, 1.0, 0.0)
        out[...] = jnp.concatenate([acc, tail], axis=1)


def _tc_msg(gu, heu, asu, gv, hev, asv):
    eb = pl.BlockSpec((EB, HP), lambda i: (i, 0))
    hb = pl.BlockSpec((EB, 16), lambda i: (i, 0))
    ab = pl.BlockSpec((HP, NT * HP), lambda i: (0, 0))
    mb = pl.BlockSpec((EB, MW), lambda i: (i, 0))
    return pl.pallas_call(
        _msg_body,
        grid=(E // EB,),
        in_specs=[eb, hb, ab, eb, hb, ab],
        out_specs=[mb, mb],
        out_shape=[jax.ShapeDtypeStruct((E, MW), _f32),
                   jax.ShapeDtypeStruct((E, MW), _f32)],
    )(gu, heu, asu, gv, hev, asv)


_ROWS_PER_TILE = N // 16


@functools.partial(
    pl.kernel, mesh=_SC_MESH,
    out_type=[jax.ShapeDtypeStruct((2, N, MW), _f32),
              jax.ShapeDtypeStruct((2, N, MW), _f32)],
    scratch_types=[pltpu.VMEM((CHUNK,), jnp.int32),
                   pltpu.VMEM((CHUNK, MW), _f32),
                   pltpu.VMEM_SHARED((N, MW), _f32),
                   pltpu.VMEM_SHARED((N, MW), _f32),
                   pltpu.SemaphoreType.DMA],
    compiler_params=pltpu.CompilerParams(use_tc_tiling_on_sc=False),
)
def _sc_scatter(msg_u, msg_v, dst_u, dst_v, zeros, pu, pv,
                idx_v, msg_v_buf, sh_u, sh_v, sem):
    cid = lax.axis_index("c")
    sid = lax.axis_index("s")
    rbase = sid * _ROWS_PER_TILE
    pltpu.sync_copy(zeros.at[pl.ds(rbase, _ROWS_PER_TILE)],
                    sh_u.at[pl.ds(rbase, _ROWS_PER_TILE)])
    pltpu.sync_copy(zeros.at[pl.ds(rbase, _ROWS_PER_TILE)],
                    sh_v.at[pl.ds(rbase, _ROWS_PER_TILE)])
    plsc.subcore_barrier()
    for msg, dst, sh in ((msg_u, dst_u, sh_u), (msg_v, dst_v, sh_v)):
        for ch in range(EPW // CHUNK):
            base = cid * (E // 2) + sid * EPW + ch * CHUNK
            pltpu.sync_copy(dst.at[pl.ds(base, CHUNK)], idx_v)
            pltpu.sync_copy(msg.at[pl.ds(base, CHUNK)], msg_v_buf)
            pltpu.sync_copy(msg_v_buf, sh.at[idx_v], add=True)
    plsc.subcore_barrier()
    pltpu.sync_copy(sh_u.at[pl.ds(rbase, _ROWS_PER_TILE)],
                    pu.at[cid, pl.ds(rbase, _ROWS_PER_TILE)])
    pltpu.sync_copy(sh_v.at[pl.ds(rbase, _ROWS_PER_TILE)],
                    pv.at[cid, pl.ds(rbase, _ROWS_PER_TILE)])


def _update_body(pu, pv, ou, ov, rwu, cbu, mwu, mbu, rwv, cbv, mwv, mbv,
                 nu, nv):
    for p, o, rw, cb, mw, mb, out in (
            (pu, ou, rwu, cbu, mwu, mbu, nu),
            (pv, ov, rwv, cbv, mwv, mbv, nv)):
        aggw = p[0] + p[1]
        deg = jnp.clip(aggw[:, HP:HP + 1], 1.0, None)
        agg = aggw[:, :HP] / deg
        oo = o[...]
        m = jax.nn.relu(
            agg + jnp.dot(oo, rw[...], preferred_element_type=_f32) + cb[...])
        cat = jnp.concatenate([m, oo], axis=1)
        out[...] = jnp.dot(cat, mw[...].T, preferred_element_type=_f32) + mb[...]


def _tc_update(pu, pv, ou, ov, wu, wv):
    return pl.pallas_call(
        _update_body,
        out_shape=[jax.ShapeDtypeStruct((N, HP), _f32),
                   jax.ShapeDtypeStruct((N, HP), _f32)],
    )(pu, pv, ou, ov, wu['root_w'], wu['conv_b'], wu['msg_w'], wu['msg_b'],
      wv['root_w'], wv['conv_b'], wv['msg_w'], wv['msg_b'])


def _finalize_body(ou, xu, ov, xv, un, vn):
    for o, x, out in ((ou, xu, un), (ov, xv, vn)):
        raw = o[...] + x[...]
        nrm = jnp.sqrt(jnp.sum(raw * raw, axis=1, keepdims=True))
        out[...] = raw / jnp.clip(nrm, 1e-12, None)


def _tc_finalize(ou, xu, ov, xv):
    return pl.pallas_call(
        _finalize_body,
        out_shape=[jax.ShapeDtypeStruct((N, HP), _f32),
                   jax.ShapeDtypeStruct((N, HP), _f32)],
    )(ou, xu, ov, xv)


def _inter_body(un, vn, bu, bv, ret, up, vp):
    i = pl.program_id(0)
    j = pl.program_id(1)

    @pl.when(jnp.logical_and(i == 0, j == 0))
    def _():
        up[...] = jnp.zeros_like(up)
        vp[...] = jnp.zeros_like(vp)

    ublk = un[...]
    vblk = vn[...]
    bui = bu[...][:, 0]
    bvj = bv[...][:, 0]
    tile = lax.dot_general(ublk, vblk, (((1,), (1,)), ((), ())),
                           preferred_element_type=_f32)
    mask = (bui[:, None] == bvj[None, :]).astype(_f32)
    tile = tile * mask
    ret[...] = tile
    up[pl.ds(i * TILE, TILE), :] += jnp.dot(
        tile, vblk, preferred_element_type=_f32)
    vp[pl.ds(j * TILE, TILE), :] += lax.dot_general(
        tile, ublk, (((0,), (0,)), ((), ())), preferred_element_type=_f32)


def _tc_inter(un, vn, bu2, bv2):
    nb = N // TILE
    return pl.pallas_call(
        _inter_body,
        grid=(nb, nb),
        in_specs=[pl.BlockSpec((TILE, HP), lambda i, j: (i, 0)),
                  pl.BlockSpec((TILE, HP), lambda i, j: (j, 0)),
                  pl.BlockSpec((TILE, 8), lambda i, j: (i, 0)),
                  pl.BlockSpec((TILE, 8), lambda i, j: (j, 0))],
        out_specs=[pl.BlockSpec((TILE, TILE), lambda i, j: (i, j)),
                   pl.BlockSpec((N, HP), lambda i, j: (0, 0)),
                   pl.BlockSpec((N, HP), lambda i, j: (0, 0))],
        out_shape=[jax.ShapeDtypeStruct((N, N), _f32),
                   jax.ShapeDtypeStruct((N, HP), _f32),
                   jax.ShapeDtypeStruct((N, HP), _f32)],
    )(un, vn, bu2, bv2)


def _s2s_side(xf, bidx, wih, whh, bih, bhh):
    onehot = (lax.broadcasted_iota(jnp.int32, (B, N), 0)
              == bidx[...][:, 0][None, :]).astype(_f32)
    xfv = xf[...]
    q_star = jnp.zeros((B, 2 * DP), _f32)
    h = jnp.zeros((B, DP), _f32)
    c = jnp.zeros((B, DP), _f32)
    for _ in range(S2S_STEPS):
        gates = (jnp.dot(q_star, wih[...].T, preferred_element_type=_f32)
                 + bih[...]
                 + jnp.dot(h, whh[...].T, preferred_element_type=_f32)
                 + bhh[...])
        i_ = gates[:, :DP]
        f_ = gates[:, DP:2 * DP]
        g_ = gates[:, 2 * DP:3 * DP]
        o_ = gates[:, 3 * DP:]
        c = jax.nn.sigmoid(f_) * c + jax.nn.sigmoid(i_) * jnp.tanh(g_)
        h = jax.nn.sigmoid(o_) * jnp.tanh(c)
        qb = jnp.dot(onehot.T, h, preferred_element_type=_f32)
        e = jnp.sum(xfv * qb, axis=1)
        mx = jnp.max(jnp.where(onehot > 0, e[None, :], -1e30), axis=1)
        mxb = jnp.sum(onehot * mx[:, None], axis=0)
        ex = jnp.exp(e - mxb)
        denom = jnp.sum(onehot * ex[None, :], axis=1)
        denb = jnp.sum(onehot * denom[:, None], axis=0)
        a = ex / (denb + 1e-16)
        r = jnp.dot(onehot, a[:, None] * xfv, preferred_element_type=_f32)
        q_star = jnp.concatenate([h, r], axis=1)
    return q_star


def _head_body(ufp, vfp, bu, bv, wihu, whhu, bihu, bhhu,
               wihv, whhv, bihv, bhhv, w1, b1, w2, b2, w3, pred):
    us = _s2s_side(ufp, bu, wihu, whhu, bihu, bhhu)
    vs = _s2s_side(vfp, bv, wihv, whhv, bihv, bhhv)
    ffp = jnp.concatenate([us, vs], axis=1)
    h1 = jax.nn.relu(
        jnp.dot(ffp, w1[...].T, preferred_element_type=_f32) + b1[...])
    h2 = jax.nn.relu(
        jnp.dot(h1, w2[...].T, preferred_element_type=_f32) + b2[...])
    pred[...] = jnp.dot(h2, w3[...].T, preferred_element_type=_f32)


def _tc_head(ufp, vfp, bu2, bv2, su, sv, w1, b1, w2, b2, w3):
    return pl.pallas_call(
        _head_body,
        out_shape=jax.ShapeDtypeStruct((B, DP), _f32),
    )(ufp, vfp, bu2, bv2, su[0], su[1], su[2], su[3],
      sv[0], sv[1], sv[2], sv[3], w1, b1, w2, b2, w3)


def kernel(solute_x, solute_edge_index, solute_edge_attr, solute_batch,
           solvent_x, solvent_edge_index, solvent_edge_attr, solvent_batch,
           params):
    pu = _prep_gather(params['solute_gather'])
    pv = _prep_gather(params['solvent_gather'])
    xu = jnp.pad(solute_x, ((0, 0), (0, HP - H)))
    xv = jnp.pad(solvent_x, ((0, 0), (0, HP - H)))
    eau = jnp.pad(solute_edge_attr, ((0, 0), (0, 16 - EI)))
    eav = jnp.pad(solvent_edge_attr, ((0, 0), (0, 16 - EI)))
    src_u, dst_u = solute_edge_index[0], solute_edge_index[1]
    src_v, dst_v = solvent_edge_index[0], solvent_edge_index[1]

    ou, ov, heu, hev = _tc_pre(xu, xv, eau, eav, pu, pv)
    zeros = jnp.zeros((N, MW), _f32)
    for _ in range(STEPS):
        gu, gv = _sc_gather(ou, ov, src_u, src_v)
        mu, mv = _tc_msg(gu, heu, pu['astack'], gv, hev, pv['astack'])
        agu, agv = _sc_scatter(mu, mv, dst_u, dst_v, zeros)
        ou, ov = _tc_update(agu, agv, ou, ov, pu, pv)

    un, vn = _tc_finalize(ou, xu, ov, xv)
    bu2 = jnp.broadcast_to(solute_batch[:, None], (N, 8)).astype(jnp.int32)
    bv2 = jnp.broadcast_to(solvent_batch[:, None], (N, 8)).astype(jnp.int32)
    ret_map, u_prime, v_prime = _tc_inter(un, vn, bu2, bv2)

    ufp = jnp.pad(jnp.concatenate([un[:, :H], u_prime[:, :H]], axis=1),
                  ((0, 0), (0, DP - 2 * H)))
    vfp = jnp.pad(jnp.concatenate([vn[:, :H], v_prime[:, :H]], axis=1),
                  ((0, 0), (0, DP - 2 * H)))
    su = _prep_s2s(params['s2s_u'])
    sv = _prep_s2s(params['s2s_v'])
    w1, w3 = _prep_pred(params)
    b1 = params['pred1_b'].reshape(1, 256)
    b2 = params['pred2_b'].reshape(1, 128)
    predf = _tc_head(ufp, vfp, bu2, bv2, su, sv, w1, b1,
                     params['pred2_w'], b2, w3)
    pred = predf[:, :1] + params['pred3_b']
    return pred, ret_map


# speedup vs baseline: 2.0208x; 1.0338x over previous
import functools

import jax
import jax.numpy as jnp
from jax import lax
from jax.experimental import pallas as pl
from jax.experimental.pallas import tpu as pltpu
from jax.experimental.pallas import tpu_sc as plsc

N, E, B, H, EI, STEPS, S2S_STEPS = 4096, 8192, 128, 52, 10, 3, 2
HP = 64
NT = EI + 1
MW = 80
EB = 2048
TILE = 256
DP = 128
NW = 32
EPW = E // NW
CHUNK = 128

_f32 = jnp.float32


def _pad2(a, shape):
    return jnp.zeros(shape, _f32).at[: a.shape[0], : a.shape[1]].set(a)


def _prep_gather(p):
    en2w = p['en2_w']
    astack = jnp.zeros((HP, NT * HP), _f32)
    for i in range(EI):
        astack = astack.at[:H, i * HP:i * HP + H].set(en2w[:, i].reshape(H, H))
    astack = astack.at[:H, EI * HP:EI * HP + H].set(p['en2_b'].reshape(H, H))
    msg_w = jnp.zeros((HP, 2 * HP), _f32)
    msg_w = msg_w.at[:H, :H].set(p['msg_w'][:, :H])
    msg_w = msg_w.at[:H, HP:HP + H].set(p['msg_w'][:, H:2 * H])
    return {
        'lin0_w': _pad2(p['lin0_w'], (HP, HP)),
        'lin0_b': jnp.zeros((1, HP), _f32).at[0, :H].set(p['lin0_b']),
        'en1_w': _pad2(p['en1_w'], (16, 16)),
        'en1_b': jnp.zeros((1, 16), _f32).at[0, :EI].set(p['en1_b']),
        'astack': astack,
        'root_w': _pad2(p['root_w'], (HP, HP)),
        'conv_b': jnp.zeros((1, HP), _f32).at[0, :H].set(p['conv_b']),
        'msg_w': msg_w,
        'msg_b': jnp.zeros((1, HP), _f32).at[0, :H].set(p['msg_b']),
    }


def _prep_s2s(p):
    d = 2 * H
    wih_p = jnp.zeros((4 * DP, 2 * DP), _f32)
    whh_p = jnp.zeros((4 * DP, DP), _f32)
    bih_p = jnp.zeros((1, 4 * DP), _f32)
    bhh_p = jnp.zeros((1, 4 * DP), _f32)
    for g in range(4):
        wih_p = wih_p.at[g * DP:g * DP + d, :d].set(p['wih'][g * d:(g + 1) * d, :d])
        wih_p = wih_p.at[g * DP:g * DP + d, DP:DP + d].set(p['wih'][g * d:(g + 1) * d, d:2 * d])
        whh_p = whh_p.at[g * DP:g * DP + d, :d].set(p['whh'][g * d:(g + 1) * d, :])
        bih_p = bih_p.at[0, g * DP:g * DP + d].set(p['bih'][g * d:(g + 1) * d])
        bhh_p = bhh_p.at[0, g * DP:g * DP + d].set(p['bhh'][g * d:(g + 1) * d])
    return wih_p, whh_p, bih_p, bhh_p


def _prep_pred(p):
    d = 2 * H
    w1 = jnp.zeros((256, 4 * DP), _f32)
    for g in range(4):
        w1 = w1.at[:, g * DP:g * DP + d].set(p['pred1_w'][:, g * d:(g + 1) * d])
    w3 = jnp.zeros((DP, DP), _f32).at[0, :].set(p['pred3_w'][0])
    return w1, w3


def _pre_body(xu, xv, eau, eav, l0wu, l0bu, l0wv, l0bv, e1wu, e1bu, e1wv, e1bv,
              outu, outv, heu, hev):
    outu[...] = jax.nn.relu(
        jnp.dot(xu[...], l0wu[...].T, preferred_element_type=_f32) + l0bu[...])
    outv[...] = jax.nn.relu(
        jnp.dot(xv[...], l0wv[...].T, preferred_element_type=_f32) + l0bv[...])
    col = lax.broadcasted_iota(jnp.int32, (E, 16), 1)
    hu = jax.nn.relu(
        jnp.dot(eau[...], e1wu[...].T, preferred_element_type=_f32) + e1bu[...])
    heu[...] = jnp.where(col == EI, 1.0, hu)
    hv = jax.nn.relu(
        jnp.dot(eav[...], e1wv[...].T, preferred_element_type=_f32) + e1bv[...])
    hev[...] = jnp.where(col == EI, 1.0, hv)


def _tc_pre(xu, xv, eau, eav, pu, pv):
    return pl.pallas_call(
        _pre_body,
        out_shape=[jax.ShapeDtypeStruct((N, HP), _f32),
                   jax.ShapeDtypeStruct((N, HP), _f32),
                   jax.ShapeDtypeStruct((E, 16), _f32),
                   jax.ShapeDtypeStruct((E, 16), _f32)],
    )(xu, xv, eau, eav, pu['lin0_w'], pu['lin0_b'], pv['lin0_w'], pv['lin0_b'],
      pu['en1_w'], pu['en1_b'], pv['en1_w'], pv['en1_b'])


_SC_MESH = plsc.VectorSubcoreMesh(core_axis_name="c", subcore_axis_name="s")


@functools.partial(
    pl.kernel, mesh=_SC_MESH,
    out_type=[jax.ShapeDtypeStruct((E, HP), _f32),
              jax.ShapeDtypeStruct((E, HP), _f32)],
    scratch_types=[pltpu.VMEM((CHUNK,), jnp.int32),
                   pltpu.VMEM((CHUNK, HP), _f32),
                   pltpu.SemaphoreType.DMA],
    compiler_params=pltpu.CompilerParams(use_tc_tiling_on_sc=False),
)
def _sc_gather(tab_u, tab_v, src_u, src_v, gu, gv, idx_v, rows_v, sem):
    cid = lax.axis_index("c")
    sid = lax.axis_index("s")
    wid = sid * 2 + cid
    for tab, src, out in ((tab_u, src_u, gu), (tab_v, src_v, gv)):
        for ch in range(EPW // CHUNK):
            base = wid * EPW + ch * CHUNK
            pltpu.sync_copy(src.at[pl.ds(base, CHUNK)], idx_v)
            pltpu.async_copy(tab.at[idx_v], rows_v, sem).wait()
            pltpu.sync_copy(rows_v, out.at[pl.ds(base, CHUNK)])


def _msg_body(gu, heu, asu, gv, hev, asv, mu, mv):
    for g, he, astk, out in ((gu, heu, asu, mu), (gv, hev, asv, mv)):
        p = jnp.dot(g[...], astk[...], preferred_element_type=_f32)
        acc = jnp.zeros((EB, HP), _f32)
        hev_ = he[...]
        for i in range(NT):
            acc = acc + hev_[:, i:i + 1] * p[:, i * HP:(i + 1) * HP]
        col = lax.broadcasted_iota(jnp.int32, (EB, 16), 1)
        tail = jnp.where(col == 0, 1.0, 0.0)
        out[...] = jnp.concatenate([acc, tail], axis=1)


def _tc_msg(gu, heu, asu, gv, hev, asv):
    eb = pl.BlockSpec((EB, HP), lambda i: (i, 0))
    hb = pl.BlockSpec((EB, 16), lambda i: (i, 0))
    ab = pl.BlockSpec((HP, NT * HP), lambda i: (0, 0))
    mb = pl.BlockSpec((EB, MW), lambda i: (i, 0))
    return pl.pallas_call(
        _msg_body,
        grid=(E // EB,),
        in_specs=[eb, hb, ab, eb, hb, ab],
        out_specs=[mb, mb],
        out_shape=[jax.ShapeDtypeStruct((E, MW), _f32),
                   jax.ShapeDtypeStruct((E, MW), _f32)],
    )(gu, heu, asu, gv, hev, asv)


_ROWS_PER_TILE = N // 16


@functools.partial(
    pl.kernel, mesh=_SC_MESH,
    out_type=[jax.ShapeDtypeStruct((2, N, MW), _f32),
              jax.ShapeDtypeStruct((2, N, MW), _f32)],
    scratch_types=[pltpu.VMEM((CHUNK,), jnp.int32),
                   pltpu.VMEM((CHUNK, MW), _f32),
                   pltpu.VMEM_SHARED((N, MW), _f32),
                   pltpu.VMEM_SHARED((N, MW), _f32),
                   pltpu.SemaphoreType.DMA],
    compiler_params=pltpu.CompilerParams(use_tc_tiling_on_sc=False),
)
def _sc_scatter(msg_u, msg_v, dst_u, dst_v, zeros, pu, pv,
                idx_v, msg_v_buf, sh_u, sh_v, sem):
    cid = lax.axis_index("c")
    sid = lax.axis_index("s")
    rbase = sid * _ROWS_PER_TILE
    pltpu.sync_copy(zeros.at[pl.ds(rbase, _ROWS_PER_TILE)],
                    sh_u.at[pl.ds(rbase, _ROWS_PER_TILE)])
    pltpu.sync_copy(zeros.at[pl.ds(rbase, _ROWS_PER_TILE)],
                    sh_v.at[pl.ds(rbase, _ROWS_PER_TILE)])
    plsc.subcore_barrier()
    for msg, dst, sh in ((msg_u, dst_u, sh_u), (msg_v, dst_v, sh_v)):
        for ch in range(EPW // CHUNK):
            base = cid * (E // 2) + sid * EPW + ch * CHUNK
            pltpu.sync_copy(dst.at[pl.ds(base, CHUNK)], idx_v)
            pltpu.sync_copy(msg.at[pl.ds(base, CHUNK)], msg_v_buf)
            pltpu.sync_copy(msg_v_buf, sh.at[idx_v], add=True)
    plsc.subcore_barrier()
    pltpu.sync_copy(sh_u.at[pl.ds(rbase, _ROWS_PER_TILE)],
                    pu.at[cid, pl.ds(rbase, _ROWS_PER_TILE)])
    pltpu.sync_copy(sh_v.at[pl.ds(rbase, _ROWS_PER_TILE)],
                    pv.at[cid, pl.ds(rbase, _ROWS_PER_TILE)])


def _update_body(final, pu, pv, ou, ov, xu, xv, rwu, cbu, mwu, mbu,
                 rwv, cbv, mwv, mbv, nu, nv):
    for p, o, x, rw, cb, mw, mb, out in (
            (pu, ou, xu, rwu, cbu, mwu, mbu, nu),
            (pv, ov, xv, rwv, cbv, mwv, mbv, nv)):
        aggw = p[0] + p[1]
        deg = jnp.clip(aggw[:, HP:HP + 1], 1.0, None)
        agg = aggw[:, :HP] / deg
        oo = o[...]
        m = jax.nn.relu(
            agg + jnp.dot(oo, rw[...], preferred_element_type=_f32) + cb[...])
        cat = jnp.concatenate([m, oo], axis=1)
        nxt = jnp.dot(cat, mw[...].T, preferred_element_type=_f32) + mb[...]
        if final:
            raw = nxt + x[...]
            nrm = jnp.sqrt(jnp.sum(raw * raw, axis=1, keepdims=True))
            nxt = raw / jnp.clip(nrm, 1e-12, None)
        out[...] = nxt


def _tc_update(final, pu, pv, ou, ov, xu, xv, wu, wv):
    return pl.pallas_call(
        functools.partial(_update_body, final),
        out_shape=[jax.ShapeDtypeStruct((N, HP), _f32),
                   jax.ShapeDtypeStruct((N, HP), _f32)],
    )(pu, pv, ou, ov, xu, xv,
      wu['root_w'], wu['conv_b'], wu['msg_w'], wu['msg_b'],
      wv['root_w'], wv['conv_b'], wv['msg_w'], wv['msg_b'])


def _inter_body(un, vn, bu, bv, ret, up, vp):
    i = pl.program_id(0)
    j = pl.program_id(1)

    @pl.when(jnp.logical_and(i == 0, j == 0))
    def _():
        up[...] = jnp.zeros_like(up)
        vp[...] = jnp.zeros_like(vp)

    bui = bu[...][:, 0]
    bvj = bv[...][:, 0]
    hit = jnp.logical_and(bui[0] <= bvj[TILE - 1], bvj[0] <= bui[TILE - 1])

    @pl.when(hit)
    def _():
        ublk = un[...]
        vblk = vn[...]
        tile = lax.dot_general(ublk, vblk, (((1,), (1,)), ((), ())),
                               preferred_element_type=_f32)
        mask = (bui[:, None] == bvj[None, :]).astype(_f32)
        tile = tile * mask
        ret[...] = tile
        up[pl.ds(i * TILE, TILE), :] += jnp.dot(
            tile, vblk, preferred_element_type=_f32)
        vp[pl.ds(j * TILE, TILE), :] += lax.dot_general(
            tile, ublk, (((0,), (0,)), ((), ())), preferred_element_type=_f32)

    @pl.when(jnp.logical_not(hit))
    def _():
        ret[...] = jnp.zeros_like(ret)


def _tc_inter(un, vn, bu2, bv2):
    nb = N // TILE
    return pl.pallas_call(
        _inter_body,
        grid=(nb, nb),
        in_specs=[pl.BlockSpec((TILE, HP), lambda i, j: (i, 0)),
                  pl.BlockSpec((TILE, HP), lambda i, j: (j, 0)),
                  pl.BlockSpec((TILE, 8), lambda i, j: (i, 0)),
                  pl.BlockSpec((TILE, 8), lambda i, j: (j, 0))],
        out_specs=[pl.BlockSpec((TILE, TILE), lambda i, j: (i, j)),
                   pl.BlockSpec((N, HP), lambda i, j: (0, 0)),
                   pl.BlockSpec((N, HP), lambda i, j: (0, 0))],
        out_shape=[jax.ShapeDtypeStruct((N, N), _f32),
                   jax.ShapeDtypeStruct((N, HP), _f32),
                   jax.ShapeDtypeStruct((N, HP), _f32)],
    )(un, vn, bu2, bv2)


def _s2s_side(xf, bidx, wih, whh, bih, bhh):
    onehot = (lax.broadcasted_iota(jnp.int32, (B, N), 0)
              == bidx[...][:, 0][None, :]).astype(_f32)
    xfv = xf[...]
    q_star = jnp.zeros((B, 2 * DP), _f32)
    h = jnp.zeros((B, DP), _f32)
    c = jnp.zeros((B, DP), _f32)
    for _ in range(S2S_STEPS):
        gates = (jnp.dot(q_star, wih[...].T, preferred_element_type=_f32)
                 + bih[...]
                 + jnp.dot(h, whh[...].T, preferred_element_type=_f32)
                 + bhh[...])
        i_ = gates[:, :DP]
        f_ = gates[:, DP:2 * DP]
        g_ = gates[:, 2 * DP:3 * DP]
        o_ = gates[:, 3 * DP:]
        c = jax.nn.sigmoid(f_) * c + jax.nn.sigmoid(i_) * jnp.tanh(g_)
        h = jax.nn.sigmoid(o_) * jnp.tanh(c)
        qb = jnp.dot(onehot.T, h, preferred_element_type=_f32)
        e = jnp.sum(xfv * qb, axis=1)
        mx = jnp.max(jnp.where(onehot > 0, e[None, :], -1e30), axis=1)
        mxb = jnp.sum(onehot * mx[:, None], axis=0)
        ex = jnp.exp(e - mxb)
        denom = jnp.sum(onehot * ex[None, :], axis=1)
        denb = jnp.sum(onehot * denom[:, None], axis=0)
        a = ex / (denb + 1e-16)
        r = jnp.dot(onehot, a[:, None] * xfv, preferred_element_type=_f32)
        q_star = jnp.concatenate([h, r], axis=1)
    return q_star


def _head_body(ufp, vfp, bu, bv, wihu, whhu, bihu, bhhu,
               wihv, whhv, bihv, bhhv, w1, b1, w2, b2, w3, pred):
    us = _s2s_side(ufp, bu, wihu, whhu, bihu, bhhu)
    vs = _s2s_side(vfp, bv, wihv, whhv, bihv, bhhv)
    ffp = jnp.concatenate([us, vs], axis=1)
    h1 = jax.nn.relu(
        jnp.dot(ffp, w1[...].T, preferred_element_type=_f32) + b1[...])
    h2 = jax.nn.relu(
        jnp.dot(h1, w2[...].T, preferred_element_type=_f32) + b2[...])
    pred[...] = jnp.dot(h2, w3[...].T, preferred_element_type=_f32)


def _tc_head(ufp, vfp, bu2, bv2, su, sv, w1, b1, w2, b2, w3):
    return pl.pallas_call(
        _head_body,
        out_shape=jax.ShapeDtypeStruct((B, DP), _f32),
    )(ufp, vfp, bu2, bv2, su[0], su[1], su[2], su[3],
      sv[0], sv[1], sv[2], sv[3], w1, b1, w2, b2, w3)


def kernel(solute_x, solute_edge_index, solute_edge_attr, solute_batch,
           solvent_x, solvent_edge_index, solvent_edge_attr, solvent_batch,
           params):
    pu = _prep_gather(params['solute_gather'])
    pv = _prep_gather(params['solvent_gather'])
    xu = jnp.pad(solute_x, ((0, 0), (0, HP - H)))
    xv = jnp.pad(solvent_x, ((0, 0), (0, HP - H)))
    eau = jnp.pad(solute_edge_attr, ((0, 0), (0, 16 - EI)))
    eav = jnp.pad(solvent_edge_attr, ((0, 0), (0, 16 - EI)))
    src_u, dst_u = solute_edge_index[0], solute_edge_index[1]
    src_v, dst_v = solvent_edge_index[0], solvent_edge_index[1]

    ou, ov, heu, hev = _tc_pre(xu, xv, eau, eav, pu, pv)
    zeros = jnp.zeros((N, MW), _f32)
    for s in range(STEPS):
        gu, gv = _sc_gather(ou, ov, src_u, src_v)
        mu, mv = _tc_msg(gu, heu, pu['astack'], gv, hev, pv['astack'])
        agu, agv = _sc_scatter(mu, mv, dst_u, dst_v, zeros)
        ou, ov = _tc_update(s == STEPS - 1, agu, agv, ou, ov, xu, xv, pu, pv)

    un, vn = ou, ov
    bu2 = jnp.broadcast_to(solute_batch[:, None], (N, 8)).astype(jnp.int32)
    bv2 = jnp.broadcast_to(solvent_batch[:, None], (N, 8)).astype(jnp.int32)
    ret_map, u_prime, v_prime = _tc_inter(un, vn, bu2, bv2)

    ufp = jnp.pad(jnp.concatenate([un[:, :H], u_prime[:, :H]], axis=1),
                  ((0, 0), (0, DP - 2 * H)))
    vfp = jnp.pad(jnp.concatenate([vn[:, :H], v_prime[:, :H]], axis=1),
                  ((0, 0), (0, DP - 2 * H)))
    su = _prep_s2s(params['s2s_u'])
    sv = _prep_s2s(params['s2s_v'])
    w1, w3 = _prep_pred(params)
    b1 = params['pred1_b'].reshape(1, 256)
    b2 = params['pred2_b'].reshape(1, 128)
    predf = _tc_head(ufp, vfp, bu2, bv2, su, sv, w1, b1,
                     params['pred2_w'], b2, w3)
    pred = predf[:, :1] + params['pred3_b']
    return pred, ret_map


# speedup vs baseline: 2.0314x; 1.0053x over previous
import functools

import jax
import jax.numpy as jnp
from jax import lax
from jax.experimental import pallas as pl
from jax.experimental.pallas import tpu as pltpu
from jax.experimental.pallas import tpu_sc as plsc

N, E, B, H, EI, STEPS, S2S_STEPS = 4096, 8192, 128, 52, 10, 3, 2
HP = 64
NT = EI + 1
MW = 80
EB = 2048
TILE = 256
DP = 128
NW = 32
EPW = E // NW
CHUNK = 128

_f32 = jnp.float32


def _pad2(a, shape):
    return jnp.zeros(shape, _f32).at[: a.shape[0], : a.shape[1]].set(a)


def _prep_gather(p):
    en2w = p['en2_w']
    astack = jnp.zeros((HP, NT * HP), _f32)
    for i in range(EI):
        astack = astack.at[:H, i * HP:i * HP + H].set(en2w[:, i].reshape(H, H))
    astack = astack.at[:H, EI * HP:EI * HP + H].set(p['en2_b'].reshape(H, H))
    msg_w = jnp.zeros((HP, 2 * HP), _f32)
    msg_w = msg_w.at[:H, :H].set(p['msg_w'][:, :H])
    msg_w = msg_w.at[:H, HP:HP + H].set(p['msg_w'][:, H:2 * H])
    return {
        'lin0_w': _pad2(p['lin0_w'], (HP, HP)),
        'lin0_b': jnp.zeros((1, HP), _f32).at[0, :H].set(p['lin0_b']),
        'en1_w': _pad2(p['en1_w'], (16, 16)),
        'en1_b': jnp.zeros((1, 16), _f32).at[0, :EI].set(p['en1_b']),
        'astack': astack,
        'root_w': _pad2(p['root_w'], (HP, HP)),
        'conv_b': jnp.zeros((1, HP), _f32).at[0, :H].set(p['conv_b']),
        'msg_w': msg_w,
        'msg_b': jnp.zeros((1, HP), _f32).at[0, :H].set(p['msg_b']),
    }


def _prep_s2s(p):
    d = 2 * H
    wih_p = jnp.zeros((4 * DP, 2 * DP), _f32)
    whh_p = jnp.zeros((4 * DP, DP), _f32)
    bih_p = jnp.zeros((1, 4 * DP), _f32)
    bhh_p = jnp.zeros((1, 4 * DP), _f32)
    for g in range(4):
        wih_p = wih_p.at[g * DP:g * DP + d, :d].set(p['wih'][g * d:(g + 1) * d, :d])
        wih_p = wih_p.at[g * DP:g * DP + d, DP:DP + d].set(p['wih'][g * d:(g + 1) * d, d:2 * d])
        whh_p = whh_p.at[g * DP:g * DP + d, :d].set(p['whh'][g * d:(g + 1) * d, :])
        bih_p = bih_p.at[0, g * DP:g * DP + d].set(p['bih'][g * d:(g + 1) * d])
        bhh_p = bhh_p.at[0, g * DP:g * DP + d].set(p['bhh'][g * d:(g + 1) * d])
    return wih_p, whh_p, bih_p, bhh_p


def _prep_pred(p):
    d = 2 * H
    w1 = jnp.zeros((256, 4 * DP), _f32)
    for g in range(4):
        w1 = w1.at[:, g * DP:g * DP + d].set(p['pred1_w'][:, g * d:(g + 1) * d])
    w3 = jnp.zeros((DP, DP), _f32).at[0, :].set(p['pred3_w'][0])
    return w1, w3


def _pre_body(xu, xv, eau, eav, l0wu, l0bu, l0wv, l0bv, e1wu, e1bu, e1wv, e1bv,
              outu, outv, heu, hev):
    outu[...] = jax.nn.relu(
        jnp.dot(xu[...], l0wu[...].T, preferred_element_type=_f32) + l0bu[...])
    outv[...] = jax.nn.relu(
        jnp.dot(xv[...], l0wv[...].T, preferred_element_type=_f32) + l0bv[...])
    col = lax.broadcasted_iota(jnp.int32, (E, 16), 1)
    hu = jax.nn.relu(
        jnp.dot(eau[...], e1wu[...].T, preferred_element_type=_f32) + e1bu[...])
    heu[...] = jnp.where(col == EI, 1.0, hu)
    hv = jax.nn.relu(
        jnp.dot(eav[...], e1wv[...].T, preferred_element_type=_f32) + e1bv[...])
    hev[...] = jnp.where(col == EI, 1.0, hv)


def _tc_pre(xu, xv, eau, eav, pu, pv):
    return pl.pallas_call(
        _pre_body,
        out_shape=[jax.ShapeDtypeStruct((N, HP), _f32),
                   jax.ShapeDtypeStruct((N, HP), _f32),
                   jax.ShapeDtypeStruct((E, 16), _f32),
                   jax.ShapeDtypeStruct((E, 16), _f32)],
    )(xu, xv, eau, eav, pu['lin0_w'], pu['lin0_b'], pv['lin0_w'], pv['lin0_b'],
      pu['en1_w'], pu['en1_b'], pv['en1_w'], pv['en1_b'])


_SC_MESH = plsc.VectorSubcoreMesh(core_axis_name="c", subcore_axis_name="s")


@functools.partial(
    pl.kernel, mesh=_SC_MESH,
    out_type=jax.ShapeDtypeStruct((E, HP), _f32),
    scratch_types=[pltpu.VMEM((CHUNK,), jnp.int32),
                   pltpu.VMEM((CHUNK, HP), _f32),
                   pltpu.SemaphoreType.DMA],
    compiler_params=pltpu.CompilerParams(use_tc_tiling_on_sc=False),
)
def _sc_gather(tab, src, out, idx_v, rows_v, sem):
    cid = lax.axis_index("c")
    sid = lax.axis_index("s")
    wid = sid * 2 + cid
    for ch in range(EPW // CHUNK):
        base = wid * EPW + ch * CHUNK
        pltpu.sync_copy(src.at[pl.ds(base, CHUNK)], idx_v)
        pltpu.async_copy(tab.at[idx_v], rows_v, sem).wait()
        pltpu.sync_copy(rows_v, out.at[pl.ds(base, CHUNK)])


def _msg_body(g, he, astk, out):
    p = jnp.dot(g[...], astk[...], preferred_element_type=_f32)
    acc = jnp.zeros((EB, HP), _f32)
    hev_ = he[...]
    for i in range(NT):
        acc = acc + hev_[:, i:i + 1] * p[:, i * HP:(i + 1) * HP]
    col = lax.broadcasted_iota(jnp.int32, (EB, 16), 1)
    tail = jnp.where(col == 0, 1.0, 0.0)
    out[...] = jnp.concatenate([acc, tail], axis=1)


def _tc_msg(g, he, astk):
    eb = pl.BlockSpec((EB, HP), lambda i: (i, 0))
    hb = pl.BlockSpec((EB, 16), lambda i: (i, 0))
    ab = pl.BlockSpec((HP, NT * HP), lambda i: (0, 0))
    mb = pl.BlockSpec((EB, MW), lambda i: (i, 0))
    return pl.pallas_call(
        _msg_body,
        grid=(E // EB,),
        in_specs=[eb, hb, ab],
        out_specs=mb,
        out_shape=jax.ShapeDtypeStruct((E, MW), _f32),
    )(g, he, astk)


_ROWS_PER_TILE = N // 16


@functools.partial(
    pl.kernel, mesh=_SC_MESH,
    out_type=jax.ShapeDtypeStruct((2, N, MW), _f32),
    scratch_types=[pltpu.VMEM((CHUNK,), jnp.int32),
                   pltpu.VMEM((CHUNK, MW), _f32),
                   pltpu.VMEM_SHARED((N, MW), _f32),
                   pltpu.SemaphoreType.DMA],
    compiler_params=pltpu.CompilerParams(use_tc_tiling_on_sc=False),
)
def _sc_scatter(msg, dst, zeros, part, idx_v, msg_v_buf, sh, sem):
    cid = lax.axis_index("c")
    sid = lax.axis_index("s")
    rbase = sid * _ROWS_PER_TILE
    pltpu.sync_copy(zeros.at[pl.ds(rbase, _ROWS_PER_TILE)],
                    sh.at[pl.ds(rbase, _ROWS_PER_TILE)])
    plsc.subcore_barrier()
    for ch in range(EPW // CHUNK):
        base = cid * (E // 2) + sid * EPW + ch * CHUNK
        pltpu.sync_copy(dst.at[pl.ds(base, CHUNK)], idx_v)
        pltpu.sync_copy(msg.at[pl.ds(base, CHUNK)], msg_v_buf)
        pltpu.sync_copy(msg_v_buf, sh.at[idx_v], add=True)
    plsc.subcore_barrier()
    pltpu.sync_copy(sh.at[pl.ds(rbase, _ROWS_PER_TILE)],
                    part.at[cid, pl.ds(rbase, _ROWS_PER_TILE)])


def _update_body(final, p, o, x, rw, cb, mw, mb, out):
    aggw = p[0] + p[1]
    deg = jnp.clip(aggw[:, HP:HP + 1], 1.0, None)
    agg = aggw[:, :HP] / deg
    oo = o[...]
    m = jax.nn.relu(
        agg + jnp.dot(oo, rw[...], preferred_element_type=_f32) + cb[...])
    cat = jnp.concatenate([m, oo], axis=1)
    nxt = jnp.dot(cat, mw[...].T, preferred_element_type=_f32) + mb[...]
    if final:
        raw = nxt + x[...]
        nrm = jnp.sqrt(jnp.sum(raw * raw, axis=1, keepdims=True))
        nxt = raw / jnp.clip(nrm, 1e-12, None)
    out[...] = nxt


def _tc_update(final, p, o, x, w):
    return pl.pallas_call(
        functools.partial(_update_body, final),
        out_shape=jax.ShapeDtypeStruct((N, HP), _f32),
    )(p, o, x, w['root_w'], w['conv_b'], w['msg_w'], w['msg_b'])


def _inter_body(un, vn, bu, bv, ret, up, vp):
    i = pl.program_id(0)
    j = pl.program_id(1)

    @pl.when(jnp.logical_and(i == 0, j == 0))
    def _():
        up[...] = jnp.zeros_like(up)
        vp[...] = jnp.zeros_like(vp)

    bui = bu[...][:, 0]
    bvj = bv[...][:, 0]
    hit = jnp.logical_and(bui[0] <= bvj[TILE - 1], bvj[0] <= bui[TILE - 1])

    @pl.when(hit)
    def _():
        ublk = un[...]
        vblk = vn[...]
        tile = lax.dot_general(ublk, vblk, (((1,), (1,)), ((), ())),
                               preferred_element_type=_f32)
        mask = (bui[:, None] == bvj[None, :]).astype(_f32)
        tile = tile * mask
        ret[...] = tile
        up[pl.ds(i * TILE, TILE), :] += jnp.dot(
            tile, vblk, preferred_element_type=_f32)
        vp[pl.ds(j * TILE, TILE), :] += lax.dot_general(
            tile, ublk, (((0,), (0,)), ((), ())), preferred_element_type=_f32)

    @pl.when(jnp.logical_not(hit))
    def _():
        ret[...] = jnp.zeros_like(ret)


def _tc_inter(un, vn, bu2, bv2):
    nb = N // TILE
    return pl.pallas_call(
        _inter_body,
        grid=(nb, nb),
        in_specs=[pl.BlockSpec((TILE, HP), lambda i, j: (i, 0)),
                  pl.BlockSpec((TILE, HP), lambda i, j: (j, 0)),
                  pl.BlockSpec((TILE, 8), lambda i, j: (i, 0)),
                  pl.BlockSpec((TILE, 8), lambda i, j: (j, 0))],
        out_specs=[pl.BlockSpec((TILE, TILE), lambda i, j: (i, j)),
                   pl.BlockSpec((N, HP), lambda i, j: (0, 0)),
                   pl.BlockSpec((N, HP), lambda i, j: (0, 0))],
        out_shape=[jax.ShapeDtypeStruct((N, N), _f32),
                   jax.ShapeDtypeStruct((N, HP), _f32),
                   jax.ShapeDtypeStruct((N, HP), _f32)],
    )(un, vn, bu2, bv2)


def _s2s_side(xf, bidx, wih, whh, bih, bhh):
    onehot = (lax.broadcasted_iota(jnp.int32, (B, N), 0)
              == bidx[...][:, 0][None, :]).astype(_f32)
    xfv = xf[...]
    q_star = jnp.zeros((B, 2 * DP), _f32)
    h = jnp.zeros((B, DP), _f32)
    c = jnp.zeros((B, DP), _f32)
    for _ in range(S2S_STEPS):
        gates = (jnp.dot(q_star, wih[...].T, preferred_element_type=_f32)
                 + bih[...]
                 + jnp.dot(h, whh[...].T, preferred_element_type=_f32)
                 + bhh[...])
        i_ = gates[:, :DP]
        f_ = gates[:, DP:2 * DP]
        g_ = gates[:, 2 * DP:3 * DP]
        o_ = gates[:, 3 * DP:]
        c = jax.nn.sigmoid(f_) * c + jax.nn.sigmoid(i_) * jnp.tanh(g_)
        h = jax.nn.sigmoid(o_) * jnp.tanh(c)
        qb = jnp.dot(onehot.T, h, preferred_element_type=_f32)
        e = jnp.sum(xfv * qb, axis=1)
        mx = jnp.max(jnp.where(onehot > 0, e[None, :], -1e30), axis=1)
        mxb = jnp.sum(onehot * mx[:, None], axis=0)
        ex = jnp.exp(e - mxb)
        denom = jnp.sum(onehot * ex[None, :], axis=1)
        denb = jnp.sum(onehot * denom[:, None], axis=0)
        a = ex / (denb + 1e-16)
        r = jnp.dot(onehot, a[:, None] * xfv, preferred_element_type=_f32)
        q_star = jnp.concatenate([h, r], axis=1)
    return q_star


def _head_body(ufp, vfp, bu, bv, wihu, whhu, bihu, bhhu,
               wihv, whhv, bihv, bhhv, w1, b1, w2, b2, w3, pred):
    us = _s2s_side(ufp, bu, wihu, whhu, bihu, bhhu)
    vs = _s2s_side(vfp, bv, wihv, whhv, bihv, bhhv)
    ffp = jnp.concatenate([us, vs], axis=1)
    h1 = jax.nn.relu(
        jnp.dot(ffp, w1[...].T, preferred_element_type=_f32) + b1[...])
    h2 = jax.nn.relu(
        jnp.dot(h1, w2[...].T, preferred_element_type=_f32) + b2[...])
    pred[...] = jnp.dot(h2, w3[...].T, preferred_element_type=_f32)


def _tc_head(ufp, vfp, bu2, bv2, su, sv, w1, b1, w2, b2, w3):
    return pl.pallas_call(
        _head_body,
        out_shape=jax.ShapeDtypeStruct((B, DP), _f32),
    )(ufp, vfp, bu2, bv2, su[0], su[1], su[2], su[3],
      sv[0], sv[1], sv[2], sv[3], w1, b1, w2, b2, w3)


def kernel(solute_x, solute_edge_index, solute_edge_attr, solute_batch,
           solvent_x, solvent_edge_index, solvent_edge_attr, solvent_batch,
           params):
    pu = _prep_gather(params['solute_gather'])
    pv = _prep_gather(params['solvent_gather'])
    xu = jnp.pad(solute_x, ((0, 0), (0, HP - H)))
    xv = jnp.pad(solvent_x, ((0, 0), (0, HP - H)))
    eau = jnp.pad(solute_edge_attr, ((0, 0), (0, 16 - EI)))
    eav = jnp.pad(solvent_edge_attr, ((0, 0), (0, 16 - EI)))
    src_u, dst_u = solute_edge_index[0], solute_edge_index[1]
    src_v, dst_v = solvent_edge_index[0], solvent_edge_index[1]

    ou, ov, heu, hev = _tc_pre(xu, xv, eau, eav, pu, pv)
    zeros = jnp.zeros((N, MW), _f32)
    for s in range(STEPS):
        fin = s == STEPS - 1
        gu = _sc_gather(ou, src_u)
        mu = _tc_msg(gu, heu, pu['astack'])
        gv = _sc_gather(ov, src_v)
        agu = _sc_scatter(mu, dst_u, zeros)
        mv = _tc_msg(gv, hev, pv['astack'])
        ou = _tc_update(fin, agu, ou, xu, pu)
        agv = _sc_scatter(mv, dst_v, zeros)
        ov = _tc_update(fin, agv, ov, xv, pv)

    un, vn = ou, ov
    bu2 = jnp.broadcast_to(solute_batch[:, None], (N, 8)).astype(jnp.int32)
    bv2 = jnp.broadcast_to(solvent_batch[:, None], (N, 8)).astype(jnp.int32)
    ret_map, u_prime, v_prime = _tc_inter(un, vn, bu2, bv2)

    ufp = jnp.pad(jnp.concatenate([un[:, :H], u_prime[:, :H]], axis=1),
                  ((0, 0), (0, DP - 2 * H)))
    vfp = jnp.pad(jnp.concatenate([vn[:, :H], v_prime[:, :H]], axis=1),
                  ((0, 0), (0, DP - 2 * H)))
    su = _prep_s2s(params['s2s_u'])
    sv = _prep_s2s(params['s2s_v'])
    w1, w3 = _prep_pred(params)
    b1 = params['pred1_b'].reshape(1, 256)
    b2 = params['pred2_b'].reshape(1, 128)
    predf = _tc_head(ufp, vfp, bu2, bv2, su, sv, w1, b1,
                     params['pred2_w'], b2, w3)
    pred = predf[:, :1] + params['pred3_b']
    return pred, ret_map


# speedup vs baseline: 2.3604x; 1.1620x over previous
import functools

import jax
import jax.numpy as jnp
from jax import lax
from jax.experimental import pallas as pl
from jax.experimental.pallas import tpu as pltpu
from jax.experimental.pallas import tpu_sc as plsc

N, E, B, H, EI, STEPS, S2S_STEPS = 4096, 8192, 128, 52, 10, 3, 2
HP = 64
NT = EI + 1
MW = 80
EB = 4096
TILE = 512
DP = 128
NW = 32
EPW = E // NW
CHUNK = 128

_f32 = jnp.float32


def _pad2(a, shape):
    return jnp.zeros(shape, _f32).at[: a.shape[0], : a.shape[1]].set(a)


def _prep_gather(p):
    en2w = p['en2_w']
    astack = jnp.zeros((HP, NT * HP), _f32)
    for i in range(EI):
        astack = astack.at[:H, i * HP:i * HP + H].set(en2w[:, i].reshape(H, H))
    astack = astack.at[:H, EI * HP:EI * HP + H].set(p['en2_b'].reshape(H, H))
    msg_w = jnp.zeros((HP, 2 * HP), _f32)
    msg_w = msg_w.at[:H, :H].set(p['msg_w'][:, :H])
    msg_w = msg_w.at[:H, HP:HP + H].set(p['msg_w'][:, H:2 * H])
    return {
        'lin0_w': _pad2(p['lin0_w'], (HP, HP)),
        'lin0_b': jnp.zeros((1, HP), _f32).at[0, :H].set(p['lin0_b']),
        'en1_w': _pad2(p['en1_w'], (16, 16)),
        'en1_b': jnp.zeros((1, 16), _f32).at[0, :EI].set(p['en1_b']),
        'astack': astack,
        'root_w': _pad2(p['root_w'], (HP, HP)),
        'conv_b': jnp.zeros((1, HP), _f32).at[0, :H].set(p['conv_b']),
        'msg_w': msg_w,
        'msg_b': jnp.zeros((1, HP), _f32).at[0, :H].set(p['msg_b']),
    }


def _prep_s2s(p):
    d = 2 * H
    wih_p = jnp.zeros((4 * DP, 2 * DP), _f32)
    whh_p = jnp.zeros((4 * DP, DP), _f32)
    bih_p = jnp.zeros((1, 4 * DP), _f32)
    bhh_p = jnp.zeros((1, 4 * DP), _f32)
    for g in range(4):
        wih_p = wih_p.at[g * DP:g * DP + d, :d].set(p['wih'][g * d:(g + 1) * d, :d])
        wih_p = wih_p.at[g * DP:g * DP + d, DP:DP + d].set(p['wih'][g * d:(g + 1) * d, d:2 * d])
        whh_p = whh_p.at[g * DP:g * DP + d, :d].set(p['whh'][g * d:(g + 1) * d, :])
        bih_p = bih_p.at[0, g * DP:g * DP + d].set(p['bih'][g * d:(g + 1) * d])
        bhh_p = bhh_p.at[0, g * DP:g * DP + d].set(p['bhh'][g * d:(g + 1) * d])
    return wih_p, whh_p, bih_p, bhh_p


def _prep_pred(p):
    d = 2 * H
    w1 = jnp.zeros((256, 4 * DP), _f32)
    for g in range(4):
        w1 = w1.at[:, g * DP:g * DP + d].set(p['pred1_w'][:, g * d:(g + 1) * d])
    w3 = jnp.zeros((DP, DP), _f32).at[0, :].set(p['pred3_w'][0])
    return w1, w3


def _pre_body(xu, xv, eau, eav, l0wu, l0bu, l0wv, l0bv, e1wu, e1bu, e1wv, e1bv,
              outu, outv, heu, hev):
    outu[...] = jax.nn.relu(
        jnp.dot(xu[...], l0wu[...].T, preferred_element_type=_f32) + l0bu[...])
    outv[...] = jax.nn.relu(
        jnp.dot(xv[...], l0wv[...].T, preferred_element_type=_f32) + l0bv[...])
    col = lax.broadcasted_iota(jnp.int32, (E, 16), 1)
    hu = jax.nn.relu(
        jnp.dot(eau[...], e1wu[...].T, preferred_element_type=_f32) + e1bu[...])
    heu[...] = jnp.where(col == EI, 1.0, hu)
    hv = jax.nn.relu(
        jnp.dot(eav[...], e1wv[...].T, preferred_element_type=_f32) + e1bv[...])
    hev[...] = jnp.where(col == EI, 1.0, hv)


def _tc_pre(xu, xv, eau, eav, pu, pv):
    return pl.pallas_call(
        _pre_body,
        out_shape=[jax.ShapeDtypeStruct((N, HP), _f32),
                   jax.ShapeDtypeStruct((N, HP), _f32),
                   jax.ShapeDtypeStruct((E, 16), _f32),
                   jax.ShapeDtypeStruct((E, 16), _f32)],
    )(xu, xv, eau, eav, pu['lin0_w'], pu['lin0_b'], pv['lin0_w'], pv['lin0_b'],
      pu['en1_w'], pu['en1_b'], pv['en1_w'], pv['en1_b'])


_SC_MESH = plsc.VectorSubcoreMesh(core_axis_name="c", subcore_axis_name="s")


@functools.partial(
    pl.kernel, mesh=_SC_MESH,
    out_type=jax.ShapeDtypeStruct((E, HP), _f32),
    scratch_types=[pltpu.VMEM((CHUNK,), jnp.int32),
                   pltpu.VMEM((CHUNK, HP), _f32),
                   pltpu.SemaphoreType.DMA],
    compiler_params=pltpu.CompilerParams(use_tc_tiling_on_sc=False),
)
def _sc_gather(tab, src, out, idx_v, rows_v, sem):
    cid = lax.axis_index("c")
    sid = lax.axis_index("s")
    wid = sid * 2 + cid
    for ch in range(EPW // CHUNK):
        base = wid * EPW + ch * CHUNK
        pltpu.sync_copy(src.at[pl.ds(base, CHUNK)], idx_v)
        pltpu.async_copy(tab.at[idx_v], rows_v, sem).wait()
        pltpu.sync_copy(rows_v, out.at[pl.ds(base, CHUNK)])


def _msg_body(g, he, astk, out):
    p = jnp.dot(g[...], astk[...], preferred_element_type=_f32)
    acc = jnp.zeros((EB, HP), _f32)
    hev_ = he[...]
    for i in range(NT):
        acc = acc + hev_[:, i:i + 1] * p[:, i * HP:(i + 1) * HP]
    col = lax.broadcasted_iota(jnp.int32, (EB, 16), 1)
    tail = jnp.where(col == 0, 1.0, 0.0)
    out[...] = jnp.concatenate([acc, tail], axis=1)


def _tc_msg(g, he, astk):
    eb = pl.BlockSpec((EB, HP), lambda i: (i, 0))
    hb = pl.BlockSpec((EB, 16), lambda i: (i, 0))
    ab = pl.BlockSpec((HP, NT * HP), lambda i: (0, 0))
    mb = pl.BlockSpec((EB, MW), lambda i: (i, 0))
    return pl.pallas_call(
        _msg_body,
        grid=(E // EB,),
        in_specs=[eb, hb, ab],
        out_specs=mb,
        out_shape=jax.ShapeDtypeStruct((E, MW), _f32),
    )(g, he, astk)


_ROWS_PER_TILE = N // 16


@functools.partial(
    pl.kernel, mesh=_SC_MESH,
    out_type=jax.ShapeDtypeStruct((2, N, MW), _f32),
    scratch_types=[pltpu.VMEM((CHUNK,), jnp.int32),
                   pltpu.VMEM((CHUNK, MW), _f32),
                   pltpu.VMEM_SHARED((N, MW), _f32),
                   pltpu.SemaphoreType.DMA],
    compiler_params=pltpu.CompilerParams(use_tc_tiling_on_sc=False),
)
def _sc_scatter(msg, dst, zeros, part, idx_v, msg_v_buf, sh, sem):
    cid = lax.axis_index("c")
    sid = lax.axis_index("s")
    rbase = sid * _ROWS_PER_TILE
    pltpu.sync_copy(zeros.at[pl.ds(rbase, _ROWS_PER_TILE)],
                    sh.at[pl.ds(rbase, _ROWS_PER_TILE)])
    plsc.subcore_barrier()
    for ch in range(EPW // CHUNK):
        base = cid * (E // 2) + sid * EPW + ch * CHUNK
        pltpu.sync_copy(dst.at[pl.ds(base, CHUNK)], idx_v)
        pltpu.sync_copy(msg.at[pl.ds(base, CHUNK)], msg_v_buf)
        pltpu.sync_copy(msg_v_buf, sh.at[idx_v], add=True)
    plsc.subcore_barrier()
    pltpu.sync_copy(sh.at[pl.ds(rbase, _ROWS_PER_TILE)],
                    part.at[cid, pl.ds(rbase, _ROWS_PER_TILE)])


def _update_body(final, p, o, x, rw, cb, mw, mb, out):
    aggw = p[0] + p[1]
    deg = jnp.clip(aggw[:, HP:HP + 1], 1.0, None)
    agg = aggw[:, :HP] / deg
    oo = o[...]
    m = jax.nn.relu(
        agg + jnp.dot(oo, rw[...], preferred_element_type=_f32) + cb[...])
    cat = jnp.concatenate([m, oo], axis=1)
    nxt = jnp.dot(cat, mw[...].T, preferred_element_type=_f32) + mb[...]
    if final:
        raw = nxt + x[...]
        nrm = jnp.sqrt(jnp.sum(raw * raw, axis=1, keepdims=True))
        nxt = raw / jnp.clip(nrm, 1e-12, None)
    out[...] = nxt


def _tc_update(final, p, o, x, w):
    return pl.pallas_call(
        functools.partial(_update_body, final),
        out_shape=jax.ShapeDtypeStruct((N, HP), _f32),
    )(p, o, x, w['root_w'], w['conv_b'], w['msg_w'], w['msg_b'])


def _inter_body(un, vn, bu, bv, ret, up, vp):
    i = pl.program_id(0)
    j = pl.program_id(1)

    @pl.when(jnp.logical_and(i == 0, j == 0))
    def _():
        up[...] = jnp.zeros_like(up)
        vp[...] = jnp.zeros_like(vp)

    bui = bu[...][:, 0]
    bvj = bv[...][:, 0]
    hit = jnp.logical_and(bui[0] <= bvj[TILE - 1], bvj[0] <= bui[TILE - 1])

    @pl.when(hit)
    def _():
        ublk = un[...]
        vblk = vn[...]
        tile = lax.dot_general(ublk, vblk, (((1,), (1,)), ((), ())),
                               preferred_element_type=_f32)
        mask = (bui[:, None] == bvj[None, :]).astype(_f32)
        tile = tile * mask
        ret[...] = tile
        up[pl.ds(i * TILE, TILE), :] += jnp.dot(
            tile, vblk, preferred_element_type=_f32)
        vp[pl.ds(j * TILE, TILE), :] += lax.dot_general(
            tile, ublk, (((0,), (0,)), ((), ())), preferred_element_type=_f32)

    @pl.when(jnp.logical_not(hit))
    def _():
        ret[...] = jnp.zeros_like(ret)


def _tc_inter(un, vn, bu2, bv2):
    nb = N // TILE
    return pl.pallas_call(
        _inter_body,
        grid=(nb, nb),
        in_specs=[pl.BlockSpec((TILE, HP), lambda i, j: (i, 0)),
                  pl.BlockSpec((TILE, HP), lambda i, j: (j, 0)),
                  pl.BlockSpec((TILE, 8), lambda i, j: (i, 0)),
                  pl.BlockSpec((TILE, 8), lambda i, j: (j, 0))],
        out_specs=[pl.BlockSpec((TILE, TILE), lambda i, j: (i, j)),
                   pl.BlockSpec((N, HP), lambda i, j: (0, 0)),
                   pl.BlockSpec((N, HP), lambda i, j: (0, 0))],
        out_shape=[jax.ShapeDtypeStruct((N, N), _f32),
                   jax.ShapeDtypeStruct((N, HP), _f32),
                   jax.ShapeDtypeStruct((N, HP), _f32)],
    )(un, vn, bu2, bv2)


def _s2s_side(xf, bidx, wih, whh, bih, bhh):
    onehot = (lax.broadcasted_iota(jnp.int32, (B, N), 0)
              == bidx[...][:, 0][None, :]).astype(_f32)
    xfv = xf[...]
    q_star = jnp.zeros((B, 2 * DP), _f32)
    h = jnp.zeros((B, DP), _f32)
    c = jnp.zeros((B, DP), _f32)
    for _ in range(S2S_STEPS):
        gates = (jnp.dot(q_star, wih[...].T, preferred_element_type=_f32)
                 + bih[...]
                 + jnp.dot(h, whh[...].T, preferred_element_type=_f32)
                 + bhh[...])
        i_ = gates[:, :DP]
        f_ = gates[:, DP:2 * DP]
        g_ = gates[:, 2 * DP:3 * DP]
        o_ = gates[:, 3 * DP:]
        c = jax.nn.sigmoid(f_) * c + jax.nn.sigmoid(i_) * jnp.tanh(g_)
        h = jax.nn.sigmoid(o_) * jnp.tanh(c)
        qb = jnp.dot(onehot.T, h, preferred_element_type=_f32)
        e = jnp.sum(xfv * qb, axis=1)
        mx = jnp.max(jnp.where(onehot > 0, e[None, :], -1e30), axis=1)
        mxb = jnp.sum(onehot * mx[:, None], axis=0)
        ex = jnp.exp(e - mxb)
        denom = jnp.sum(onehot * ex[None, :], axis=1)
        denb = jnp.sum(onehot * denom[:, None], axis=0)
        a = ex / (denb + 1e-16)
        r = jnp.dot(onehot, a[:, None] * xfv, preferred_element_type=_f32)
        q_star = jnp.concatenate([h, r], axis=1)
    return q_star


def _head_body(ufp, vfp, bu, bv, wihu, whhu, bihu, bhhu,
               wihv, whhv, bihv, bhhv, w1, b1, w2, b2, w3, pred):
    us = _s2s_side(ufp, bu, wihu, whhu, bihu, bhhu)
    vs = _s2s_side(vfp, bv, wihv, whhv, bihv, bhhv)
    ffp = jnp.concatenate([us, vs], axis=1)
    h1 = jax.nn.relu(
        jnp.dot(ffp, w1[...].T, preferred_element_type=_f32) + b1[...])
    h2 = jax.nn.relu(
        jnp.dot(h1, w2[...].T, preferred_element_type=_f32) + b2[...])
    pred[...] = jnp.dot(h2, w3[...].T, preferred_element_type=_f32)


def _tc_head(ufp, vfp, bu2, bv2, su, sv, w1, b1, w2, b2, w3):
    return pl.pallas_call(
        _head_body,
        out_shape=jax.ShapeDtypeStruct((B, DP), _f32),
    )(ufp, vfp, bu2, bv2, su[0], su[1], su[2], su[3],
      sv[0], sv[1], sv[2], sv[3], w1, b1, w2, b2, w3)


def kernel(solute_x, solute_edge_index, solute_edge_attr, solute_batch,
           solvent_x, solvent_edge_index, solvent_edge_attr, solvent_batch,
           params):
    pu = _prep_gather(params['solute_gather'])
    pv = _prep_gather(params['solvent_gather'])
    xu = jnp.pad(solute_x, ((0, 0), (0, HP - H)))
    xv = jnp.pad(solvent_x, ((0, 0), (0, HP - H)))
    eau = jnp.pad(solute_edge_attr, ((0, 0), (0, 16 - EI)))
    eav = jnp.pad(solvent_edge_attr, ((0, 0), (0, 16 - EI)))
    src_u, dst_u = solute_edge_index[0], solute_edge_index[1]
    src_v, dst_v = solvent_edge_index[0], solvent_edge_index[1]

    ou, ov, heu, hev = _tc_pre(xu, xv, eau, eav, pu, pv)
    zeros = jnp.zeros((N, MW), _f32)
    for s in range(STEPS):
        fin = s == STEPS - 1
        gu = _sc_gather(ou, src_u)
        mu = _tc_msg(gu, heu, pu['astack'])
        gv = _sc_gather(ov, src_v)
        agu = _sc_scatter(mu, dst_u, zeros)
        mv = _tc_msg(gv, hev, pv['astack'])
        ou = _tc_update(fin, agu, ou, xu, pu)
        agv = _sc_scatter(mv, dst_v, zeros)
        ov = _tc_update(fin, agv, ov, xv, pv)

    un, vn = ou, ov
    bu2 = jnp.broadcast_to(solute_batch[:, None], (N, 8)).astype(jnp.int32)
    bv2 = jnp.broadcast_to(solvent_batch[:, None], (N, 8)).astype(jnp.int32)
    ret_map, u_prime, v_prime = _tc_inter(un, vn, bu2, bv2)

    ufp = jnp.pad(jnp.concatenate([un[:, :H], u_prime[:, :H]], axis=1),
                  ((0, 0), (0, DP - 2 * H)))
    vfp = jnp.pad(jnp.concatenate([vn[:, :H], v_prime[:, :H]], axis=1),
                  ((0, 0), (0, DP - 2 * H)))
    su = _prep_s2s(params['s2s_u'])
    sv = _prep_s2s(params['s2s_v'])
    w1, w3 = _prep_pred(params)
    b1 = params['pred1_b'].reshape(1, 256)
    b2 = params['pred2_b'].reshape(1, 128)
    predf = _tc_head(ufp, vfp, bu2, bv2, su, sv, w1, b1,
                     params['pred2_w'], b2, w3)
    pred = predf[:, :1] + params['pred3_b']
    return pred, ret_map
